# baseline (speedup 1.0000x reference)
import functools

import jax
import jax.numpy as jnp
from jax import lax
from jax.experimental import pallas as pl
from jax.experimental.pallas import tpu as pltpu
from jax.experimental.pallas import tpu_sc as plsc

V = 12224
ROW = (11, 1, 256)
B = 4096
NC, NS = 2, 16
NW = NC * NS
BPW = B // NW
C = 8
NCH = BPW // C
NBUF = 5
LAG = 2

_mesh = plsc.VectorSubcoreMesh(core_axis_name="c", subcore_axis_name="s")


@functools.partial(
    pl.kernel,
    mesh=_mesh,
    out_type=jax.ShapeDtypeStruct((B,) + ROW, jnp.float32),
    scratch_types=[
        pltpu.VMEM((BPW,), jnp.int32),
    ]
    + [pltpu.VMEM((C,) + ROW, jnp.float32) for _ in range(NBUF)]
    + [pltpu.SemaphoreType.DMA for _ in range(2 * NBUF)],
)
def _sc_gather(table_hbm, idx_hbm, out_hbm, idx_v, *bufs_and_sems):
    bufs = bufs_and_sems[:NBUF]
    gsems = bufs_and_sems[NBUF : 2 * NBUF]
    osems = bufs_and_sems[2 * NBUF :]

    wid = lax.axis_index("s") * NC + lax.axis_index("c")
    base = wid * BPW
    pltpu.sync_copy(idx_hbm.at[pl.ds(base, BPW)], idx_v)

    gops = [None] * NCH
    oops = [None] * NCH
    for ci in range(NBUF - LAG):
        gops[ci] = pltpu.async_copy(
            table_hbm.at[idx_v.at[pl.ds(ci * C, C)]], bufs[ci], gsems[ci]
        )
    for ci in range(NCH):
        p = ci % NBUF
        gops[ci].wait()
        oops[ci] = pltpu.async_copy(
            bufs[p], out_hbm.at[pl.ds(base + ci * C, C)], osems[p]
        )
        tgt = ci + NBUF - LAG
        if tgt < NCH:
            if ci >= LAG:
                oops[ci - LAG].wait()
            gops[tgt] = pltpu.async_copy(
                table_hbm.at[idx_v.at[pl.ds(tgt * C, C)]],
                bufs[tgt % NBUF],
                gsems[tgt % NBUF],
            )
    for ci in range(NCH - NBUF, NCH):
        oops[ci].wait()


def kernel(table, indices):
    return _sc_gather(table, indices.astype(jnp.int32))


# speedup vs baseline: 1.0128x; 1.0128x over previous
import functools

import jax
import jax.numpy as jnp
from jax import lax
from jax.experimental import pallas as pl
from jax.experimental.pallas import tpu as pltpu
from jax.experimental.pallas import tpu_sc as plsc

V = 12224
ROW = (11, 1, 256)
B = 4096
NC, NS = 2, 16
NW = NC * NS
BPW = B // NW
C = 8
NCH = BPW // C
NBUF = 5

_mesh = plsc.VectorSubcoreMesh(core_axis_name="c", subcore_axis_name="s")


@functools.partial(
    pl.kernel,
    mesh=_mesh,
    out_type=jax.ShapeDtypeStruct((B,) + ROW, jnp.float32),
    scratch_types=[
        pltpu.VMEM((BPW,), jnp.int32),
    ]
    + [pltpu.VMEM((C,) + ROW, jnp.float32) for _ in range(NBUF)]
    + [pltpu.SemaphoreType.DMA for _ in range(2 * NBUF)],
)
def _sc_gather(table_hbm, idx_hbm, out_hbm, idx_v, *bufs_and_sems):
    bufs = bufs_and_sems[:NBUF]
    gsems = bufs_and_sems[NBUF : 2 * NBUF]
    osems = bufs_and_sems[2 * NBUF :]

    wid = lax.axis_index("s") * NC + lax.axis_index("c")
    base = wid * BPW
    pltpu.sync_copy(idx_hbm.at[pl.ds(base, BPW)], idx_v)

    gops = [None] * NCH
    oops = [None] * NCH
    for ci in range(NBUF):
        gops[ci] = pltpu.async_copy(
            table_hbm.at[idx_v.at[pl.ds(ci * C, C)]], bufs[ci], gsems[ci]
        )
    for ci in range(NCH):
        p = ci % NBUF
        gops[ci].wait()
        oops[ci] = pltpu.async_copy(
            bufs[p], out_hbm.at[pl.ds(base + ci * C, C)], osems[p]
        )
        nxt = ci + NBUF
        if nxt < NCH:
            oops[ci].wait()
            gops[nxt] = pltpu.async_copy(
                table_hbm.at[idx_v.at[pl.ds(nxt * C, C)]], bufs[p], gsems[p]
            )
    for ci in range(NCH - NBUF, NCH):
        oops[ci].wait()


def kernel(table, indices):
    return _sc_gather(table, indices.astype(jnp.int32))
